# baseline (speedup 1.0000x reference)
import jax
import jax.numpy as jnp
from jax.experimental import pallas as pl
from jax.experimental.pallas import tpu as pltpu

N_TOKENS = 8192
NUM_CLASSES = 1024
EMBED_DIM = 128
AUG = EMBED_DIM + 8

BLK = 4096
STEPS = N_TOKENS // BLK

_PREC = jax.lax.Precision.DEFAULT


def _fused_kernel(x_ref, y_ref, p_ref, c_ref, o_ref, acc_ref, ua_ref,
                  xsq_ref):
    p = pl.program_id(0)
    i = pl.program_id(1)

    @pl.when((p == 0) & (i == 0))
    def _init():
        acc_ref[...] = jnp.zeros_like(acc_ref)

    @pl.when(p == 0)
    def _accum():
        y_blk = y_ref[...]
        cls = jax.lax.broadcasted_iota(jnp.int32, (BLK, NUM_CLASSES), 1)
        oh = (y_blk == cls).astype(jnp.bfloat16)
        x = x_ref[...]
        xcat = jnp.concatenate(
            [x.astype(jnp.bfloat16),
             jnp.ones((BLK, AUG - EMBED_DIM), jnp.bfloat16)], axis=1)
        acc_ref[...] += jax.lax.dot_general(
            oh, xcat, (((0,), (0,)), ((), ())),
            precision=_PREC, preferred_element_type=jnp.float32)
        xsq_ref[pl.ds(i * BLK, BLK), :] = jax.lax.dot_general(
            x * x, jnp.ones((1, EMBED_DIM), jnp.float32),
            (((1,), (1,)), ((), ())),
            precision=_PREC, preferred_element_type=jnp.float32)

    @pl.when((p == 0) & (i == STEPS - 1))
    def _update():
        cnt = acc_ref[:, EMBED_DIM:EMBED_DIM + 1]
        sums = acc_ref[:, :EMBED_DIM]
        new = sums / jnp.maximum(cnt, 1.0)
        c = c_ref[...]
        u = jnp.where(cnt > 0.0, (c * p_ref[...] + new) / (c + 1.0),
                      p_ref[...])
        usq = jnp.sum(u * u, axis=1, keepdims=True)
        ua_ref[...] = jnp.concatenate(
            [u + u, jnp.ones((NUM_CLASSES, 1), jnp.float32), -usq,
             jnp.zeros((NUM_CLASSES, AUG - EMBED_DIM - 2), jnp.float32)],
            axis=1).astype(jnp.bfloat16)

    @pl.when(p == 1)
    def _dist():
        x = x_ref[...]
        xa = jnp.concatenate(
            [x.astype(jnp.bfloat16),
             -xsq_ref[pl.ds(i * BLK, BLK), :].astype(jnp.bfloat16),
             jnp.ones((BLK, 1), jnp.bfloat16),
             jnp.zeros((BLK, AUG - EMBED_DIM - 2), jnp.bfloat16)], axis=1)
        d = jax.lax.dot_general(xa, ua_ref[...], (((1,), (1,)), ((), ())),
                                precision=_PREC,
                                preferred_element_type=jnp.float32)
        o_ref[...] = jnp.minimum(d, 0.0)


def kernel(x, y_true, prototypes, counter):
    y2 = y_true.reshape(N_TOKENS, 1)
    c2 = counter.reshape(NUM_CLASSES, 1)
    out = pl.pallas_call(
        _fused_kernel,
        grid=(2, STEPS),
        in_specs=[
            pl.BlockSpec((BLK, EMBED_DIM), lambda p, i: (i, 0)),
            pl.BlockSpec((BLK, 1), lambda p, i: (i, 0)),
            pl.BlockSpec((NUM_CLASSES, EMBED_DIM), lambda p, i: (0, 0)),
            pl.BlockSpec((NUM_CLASSES, 1), lambda p, i: (0, 0)),
        ],
        out_specs=pl.BlockSpec((BLK, NUM_CLASSES), lambda p, i: (i * p, 0)),
        out_shape=jax.ShapeDtypeStruct((N_TOKENS, NUM_CLASSES), jnp.float32),
        scratch_shapes=[
            pltpu.VMEM((NUM_CLASSES, AUG), jnp.float32),
            pltpu.VMEM((NUM_CLASSES, AUG), jnp.bfloat16),
            pltpu.VMEM((N_TOKENS, 1), jnp.float32),
        ],
    )(x, y2, prototypes, c2)
    return out
